# Initial kernel scaffold; baseline (speedup 1.0000x reference)
import functools

import jax
import jax.numpy as jnp
from jax import lax
from jax.experimental import pallas as pl
from jax.experimental.pallas import tpu as pltpu
from jax.experimental.pallas import tpu_sc as plsc

N = 10000
NP = 10240
E = 320000
D = 128
G = 64
NC, NS = 2, 16
TILES = NC * NS
EP = TILES * 80 * 128
GPT = EP // (TILES * 128)
ROWS_PT = NP // NS
KW = 4
NWIN = GPT // KW
WROWS = KW * 128

_mesh = plsc.VectorSubcoreMesh(core_axis_name="c", subcore_axis_name="s")


@functools.partial(
    pl.kernel,
    out_type=(jax.ShapeDtypeStruct((NP,), jnp.float32),
              jax.ShapeDtypeStruct((NP,), jnp.float32)),
    mesh=_mesh,
    scratch_types=[
        pltpu.VMEM((GPT, 128), jnp.int32),
        pltpu.VMEM((128,), jnp.float32),
        pltpu.VMEM((ROWS_PT,), jnp.float32),
        pltpu.VMEM_SHARED((NP,), jnp.float32),
    ],
)
def _deg_kernel(dst2_hbm, out0, out1, dstb, onesb, zbuf, degsp):
    c = lax.axis_index("c")
    s = lax.axis_index("s")
    wid = c * NS + s
    z16 = jnp.zeros((16,), jnp.float32)
    o16 = jnp.ones((16,), jnp.float32)
    for k in range(128 // 16):
        onesb[pl.ds(k * 16, 16)] = o16
    for k in range(ROWS_PT // 16):
        zbuf[pl.ds(k * 16, 16)] = z16
    pltpu.sync_copy(zbuf, degsp.at[pl.ds(s * ROWS_PT, ROWS_PT)])
    plsc.subcore_barrier()
    pltpu.sync_copy(dst2_hbm.at[pl.ds(wid * GPT, GPT)], dstb)
    for g in range(GPT):
        pltpu.sync_copy(onesb, degsp.at[dstb.at[g]], add=True)
    plsc.subcore_barrier()

    @pl.when(c == 0)
    def _():
        pltpu.sync_copy(degsp.at[pl.ds(s * ROWS_PT, ROWS_PT)],
                        out0.at[pl.ds(s * ROWS_PT, ROWS_PT)])

    @pl.when(c == 1)
    def _():
        pltpu.sync_copy(degsp.at[pl.ds(s * ROWS_PT, ROWS_PT)],
                        out1.at[pl.ds(s * ROWS_PT, ROWS_PT)])


@functools.partial(
    pl.kernel,
    out_type=(jax.ShapeDtypeStruct((NP, D), jnp.float32),
              jax.ShapeDtypeStruct((NP, D), jnp.float32)),
    mesh=_mesh,
    scratch_types=[
        pltpu.VMEM((GPT, 128), jnp.int32),
        pltpu.VMEM((GPT, 128), jnp.int32),
        pltpu.VMEM((WROWS, D), jnp.float32),
        pltpu.VMEM_SHARED((NP, D), jnp.float32),
        pltpu.SemaphoreType.DMA,
    ],
)
def _agg_kernel(ys_hbm, src2_hbm, dst2_hbm, out0, out1,
                srcb, dstb, rowsb, accsp, sem):
    c = lax.axis_index("c")
    s = lax.axis_index("s")
    wid = c * NS + s
    rbase = s * ROWS_PT
    z16 = jnp.zeros((16,), jnp.float32)

    def zrow(i, carry):
        for k in range(D // 16):
            rowsb[i, pl.ds(k * 16, 16)] = z16
        return carry
    lax.fori_loop(0, WROWS, zrow, 0)
    pltpu.sync_copy(rowsb, accsp.at[pl.ds(rbase, WROWS)])
    pltpu.sync_copy(rowsb.at[pl.ds(0, ROWS_PT - WROWS)],
                    accsp.at[pl.ds(rbase + WROWS, ROWS_PT - WROWS)])
    plsc.subcore_barrier()

    pltpu.sync_copy(src2_hbm.at[pl.ds(wid * GPT, GPT)], srcb)
    pltpu.sync_copy(dst2_hbm.at[pl.ds(wid * GPT, GPT)], dstb)

    def win(w, carry):
        g0 = w * KW
        cps = [pltpu.async_copy(ys_hbm.at[srcb.at[g0 + j]],
                                rowsb.at[pl.ds(j * 128, 128)], sem)
               for j in range(KW)]
        for cp in cps:
            cp.wait()
        for j in range(KW):
            pltpu.sync_copy(rowsb.at[pl.ds(j * 128, 128)],
                            accsp.at[dstb.at[g0 + j]], add=True)
        return carry
    lax.fori_loop(0, NWIN, win, 0)
    plsc.subcore_barrier()

    @pl.when(c == 0)
    def _():
        pltpu.sync_copy(accsp.at[pl.ds(rbase, ROWS_PT)],
                        out0.at[pl.ds(rbase, ROWS_PT)])

    @pl.when(c == 1)
    def _():
        pltpu.sync_copy(accsp.at[pl.ds(rbase, ROWS_PT)],
                        out1.at[pl.ds(rbase, ROWS_PT)])


BR = 1024
NB = NP // BR


def _ka_body(x_ref, w_ref, d0_ref, d1_ref, ys_ref, dis_ref):
    deg = d0_ref[...] + d1_ref[...] + 1.0
    dis = lax.rsqrt(deg)
    y = jnp.dot(x_ref[...], w_ref[...], preferred_element_type=jnp.float32)
    ys_ref[...] = y * dis[:, None]
    dis_ref[...] = dis


def _ka_call(xp, W1, d0, d1):
    return pl.pallas_call(
        _ka_body,
        grid=(NB,),
        in_specs=[
            pl.BlockSpec((BR, D), lambda i: (i, 0)),
            pl.BlockSpec((D, D), lambda i: (0, 0)),
            pl.BlockSpec((BR,), lambda i: (i,)),
            pl.BlockSpec((BR,), lambda i: (i,)),
        ],
        out_specs=[
            pl.BlockSpec((BR, D), lambda i: (i, 0)),
            pl.BlockSpec((BR,), lambda i: (i,)),
        ],
        out_shape=[
            jax.ShapeDtypeStruct((NP, D), jnp.float32),
            jax.ShapeDtypeStruct((NP,), jnp.float32),
        ],
    )(xp, W1, d0, d1)


def _kb_body(a0_ref, a1_ref, ys1_ref, dis_ref, b_ref, w_ref, ys2_ref):
    dis = dis_ref[...]
    h = (a0_ref[...] + a1_ref[...] + ys1_ref[...]) * dis[:, None] \
        + b_ref[...][None, :]
    h = jnp.maximum(h, 0.0)
    y2 = jnp.dot(h, w_ref[...], preferred_element_type=jnp.float32)
    ys2_ref[...] = y2 * dis[:, None]


def _kb_call(a0, a1, ys1, dis, b1, W2):
    return pl.pallas_call(
        _kb_body,
        grid=(NB,),
        in_specs=[
            pl.BlockSpec((BR, D), lambda i: (i, 0)),
            pl.BlockSpec((BR, D), lambda i: (i, 0)),
            pl.BlockSpec((BR, D), lambda i: (i, 0)),
            pl.BlockSpec((BR,), lambda i: (i,)),
            pl.BlockSpec((D,), lambda i: (0,)),
            pl.BlockSpec((D, D), lambda i: (0, 0)),
        ],
        out_specs=pl.BlockSpec((BR, D), lambda i: (i, 0)),
        out_shape=jax.ShapeDtypeStruct((NP, D), jnp.float32),
    )(a0, a1, ys1, dis, b1, W2)


def _kc_body(a0_ref, a1_ref, ys2_ref, dis_ref, b_ref, bat_ref, wfc_ref,
             bfc_ref, out_ref, sums, cnt):
    i = pl.program_id(0)

    @pl.when(i == 0)
    def _():
        sums[...] = jnp.zeros_like(sums)
        cnt[...] = jnp.zeros_like(cnt)

    dis = dis_ref[...]
    h = (a0_ref[...] + a1_ref[...] + ys2_ref[...]) * dis[:, None] \
        + b_ref[...][None, :]
    h = jnp.maximum(h, 0.0)
    bb = bat_ref[...]
    oh = (bb[:, None] == lax.broadcasted_iota(jnp.float32, (BR, G), 1))
    oh = oh.astype(jnp.float32)
    dn = (((0,), (0,)), ((), ()))
    sums[...] += lax.dot_general(oh, h, dn,
                                 preferred_element_type=jnp.float32)
    cnt[...] += lax.dot_general(oh, jnp.ones((BR, D), jnp.float32), dn,
                                preferred_element_type=jnp.float32)

    @pl.when(i == pl.num_programs(0) - 1)
    def _():
        pooled = sums[...] / jnp.maximum(cnt[...], 1.0)
        o = jnp.dot(pooled, wfc_ref[...], preferred_element_type=jnp.float32)
        out_ref[...] = jnp.maximum(o + bfc_ref[...][None, :], 0.0)


def _kc_call(a0, a1, ys2, dis, b2, batf, Wfc, bfc):
    return pl.pallas_call(
        _kc_body,
        grid=(NB,),
        in_specs=[
            pl.BlockSpec((BR, D), lambda i: (i, 0)),
            pl.BlockSpec((BR, D), lambda i: (i, 0)),
            pl.BlockSpec((BR, D), lambda i: (i, 0)),
            pl.BlockSpec((BR,), lambda i: (i,)),
            pl.BlockSpec((D,), lambda i: (0,)),
            pl.BlockSpec((BR,), lambda i: (i,)),
            pl.BlockSpec((D, D), lambda i: (0, 0)),
            pl.BlockSpec((D,), lambda i: (0,)),
        ],
        out_specs=pl.BlockSpec((G, D), lambda i: (0, 0)),
        out_shape=jax.ShapeDtypeStruct((G, D), jnp.float32),
        scratch_shapes=[
            pltpu.VMEM((G, D), jnp.float32),
            pltpu.VMEM((G, D), jnp.float32),
        ],
    )(a0, a1, ys2, dis, b2, batf, Wfc, bfc)


def kernel(x, edge_index, batch, W1, b1, W2, b2, Wfc, bfc):
    src = edge_index[0].astype(jnp.int32)
    dst = edge_index[1].astype(jnp.int32)
    npad = EP - E
    pad_idx = N + (jnp.arange(npad, dtype=jnp.int32) % (NP - N))
    src2 = jnp.concatenate([src, pad_idx]).reshape(EP // 128, 128)
    dst2 = jnp.concatenate([dst, pad_idx]).reshape(EP // 128, 128)
    xp = jnp.pad(x, ((0, NP - N), (0, 0)))
    batf = jnp.pad(batch.astype(jnp.float32), (0, NP - N),
                   constant_values=float(G))

    d0, d1 = _deg_kernel(dst2)
    ys1, dis = _ka_call(xp, W1, d0, d1)
    a10, a11 = _agg_kernel(ys1, src2, dst2)
    ys2 = _kb_call(a10, a11, ys1, dis, b1, W2)
    a20, a21 = _agg_kernel(ys2, src2, dst2)
    return _kc_call(a20, a21, ys2, dis, b2, batf, Wfc, bfc)


# speedup vs baseline: 4.7266x; 4.7266x over previous
import functools

import jax
import jax.numpy as jnp
from jax import lax
from jax.experimental import pallas as pl
from jax.experimental.pallas import tpu as pltpu
from jax.experimental.pallas import tpu_sc as plsc

N = 10000
NP = 10240
P = 4
CH = 2816
TRASH = 128
NACC = CH + TRASH
E = 320000
D = 128
G = 64
NS = 16
EP = 327680
NGR = EP // 128
GPT = NGR // NS
ROWS_PT = NP // NS
ZR_PT = NACC // NS
KW = 4
NWIN = GPT // KW
WROWS = KW * 128

_mesh2 = plsc.VectorSubcoreMesh(core_axis_name="c", subcore_axis_name="s")
_mesh1 = plsc.VectorSubcoreMesh(core_axis_name="c", subcore_axis_name="s",
                                num_cores=1)


@functools.partial(
    pl.kernel,
    out_type=(jax.ShapeDtypeStruct((NP,), jnp.float32),
              jax.ShapeDtypeStruct((NP,), jnp.float32)),
    mesh=_mesh2,
    scratch_types=[
        pltpu.VMEM((GPT // 2, 128), jnp.int32),
        pltpu.VMEM((128,), jnp.float32),
        pltpu.VMEM((ROWS_PT,), jnp.float32),
        pltpu.VMEM_SHARED((NP,), jnp.float32),
    ],
)
def _deg_kernel(dst2_hbm, out0, out1, dstb, onesb, zbuf, degsp):
    c = lax.axis_index("c")
    s = lax.axis_index("s")
    wid = c * NS + s
    gpt = GPT // 2
    z16 = jnp.zeros((16,), jnp.float32)
    o16 = jnp.ones((16,), jnp.float32)
    for k in range(128 // 16):
        onesb[pl.ds(k * 16, 16)] = o16
    for k in range(ROWS_PT // 16):
        zbuf[pl.ds(k * 16, 16)] = z16
    pltpu.sync_copy(zbuf, degsp.at[pl.ds(s * ROWS_PT, ROWS_PT)])
    plsc.subcore_barrier()
    pltpu.sync_copy(dst2_hbm.at[pl.ds(wid * gpt, gpt)], dstb)
    for g in range(gpt):
        pltpu.sync_copy(onesb, degsp.at[dstb.at[g]], add=True)
    plsc.subcore_barrier()

    @pl.when(c == 0)
    def _():
        pltpu.sync_copy(degsp.at[pl.ds(s * ROWS_PT, ROWS_PT)],
                        out0.at[pl.ds(s * ROWS_PT, ROWS_PT)])

    @pl.when(c == 1)
    def _():
        pltpu.sync_copy(degsp.at[pl.ds(s * ROWS_PT, ROWS_PT)],
                        out1.at[pl.ds(s * ROWS_PT, ROWS_PT)])


@functools.partial(
    pl.kernel,
    out_type=jax.ShapeDtypeStruct((NP, D), jnp.float32),
    mesh=_mesh1,
    scratch_types=[
        pltpu.VMEM((GPT, 128), jnp.int32),
        pltpu.VMEM((GPT, 128), jnp.int32),
        pltpu.VMEM((WROWS, D), jnp.float32),
        pltpu.VMEM_SHARED((NP, D), jnp.float32),
        pltpu.SemaphoreType.DMA,
    ],
)
def _agg_full(ys_hbm, src2_hbm, dst2_hbm, out,
              srcb, dstb, rowsb, accsp, sem):
    s = lax.axis_index("s")
    rbase = s * ROWS_PT
    z16 = jnp.zeros((16,), jnp.float32)

    def zrow(i, carry):
        for k in range(D // 16):
            rowsb[i, pl.ds(k * 16, 16)] = z16
        return carry
    lax.fori_loop(0, WROWS, zrow, 0)
    pltpu.sync_copy(rowsb, accsp.at[pl.ds(rbase, WROWS)])
    pltpu.sync_copy(rowsb.at[pl.ds(0, ROWS_PT - WROWS)],
                    accsp.at[pl.ds(rbase + WROWS, ROWS_PT - WROWS)])
    plsc.subcore_barrier()

    pltpu.sync_copy(src2_hbm.at[pl.ds(s * GPT, GPT)], srcb)
    pltpu.sync_copy(dst2_hbm.at[pl.ds(s * GPT, GPT)], dstb)

    def win(w, carry):
        g0 = w * KW
        cps = [pltpu.async_copy(ys_hbm.at[srcb.at[g0 + j]],
                                rowsb.at[pl.ds(j * 128, 128)], sem)
               for j in range(KW)]
        for cp in cps:
            cp.wait()
        for j in range(KW):
            pltpu.sync_copy(rowsb.at[pl.ds(j * 128, 128)],
                            accsp.at[dstb.at[g0 + j]], add=True)
        return carry
    lax.fori_loop(0, NWIN, win, 0)
    plsc.subcore_barrier()
    pltpu.sync_copy(accsp.at[pl.ds(rbase, ROWS_PT)],
                    out.at[pl.ds(rbase, ROWS_PT)])


@functools.partial(
    pl.kernel,
    out_type=jax.ShapeDtypeStruct((NP, D), jnp.float32),
    mesh=_mesh1,
    scratch_types=[
        pltpu.VMEM((GPT, 128), jnp.int32),
        pltpu.VMEM((GPT, 128), jnp.int32),
        pltpu.VMEM((WROWS, D), jnp.float32),
        pltpu.VMEM_SHARED((NACC, D), jnp.float32),
        pltpu.SemaphoreType.DMA,
    ],
)
def _agg_chunk(ys_hbm, src2_hbm, dst2_hbm, out,
               srcb, dstb, rowsb, accsp, sem):
    s = lax.axis_index("s")
    z16 = jnp.zeros((16,), jnp.float32)

    def zrow(i, carry):
        for k in range(D // 16):
            rowsb[i, pl.ds(k * 16, 16)] = z16
        return carry
    lax.fori_loop(0, WROWS, zrow, 0)

    pltpu.sync_copy(src2_hbm.at[pl.ds(s * GPT, GPT)], srcb)

    for p in range(P):
        pltpu.sync_copy(rowsb.at[pl.ds(0, ZR_PT)],
                        accsp.at[pl.ds(s * ZR_PT, ZR_PT)])
        pltpu.sync_copy(dst2_hbm.at[pl.ds(s * GPT, GPT)], dstb)
        base = p * CH

        def adjrow(g, carry):
            for k in range(128 // 16):
                v = dstb[g, pl.ds(k * 16, 16)]
                inr = (v >= base) & (v < base + CH)
                tr = CH + (v & (TRASH - 1))
                dstb[g, pl.ds(k * 16, 16)] = jnp.where(inr, v - base, tr)
            return carry
        lax.fori_loop(0, GPT, adjrow, 0)
        plsc.subcore_barrier()

        def win(w, carry):
            g0 = w * KW
            cps = [pltpu.async_copy(ys_hbm.at[srcb.at[g0 + j]],
                                    rowsb.at[pl.ds(j * 128, 128)], sem)
                   for j in range(KW)]
            for cp in cps:
                cp.wait()
            for j in range(KW):
                pltpu.sync_copy(rowsb.at[pl.ds(j * 128, 128)],
                                accsp.at[dstb.at[g0 + j]], add=True)
            return carry
        lax.fori_loop(0, NWIN, win, 0)
        plsc.subcore_barrier()
        cp_pt = (CH if p < P - 1 else NP - (P - 1) * CH) // NS
        pltpu.sync_copy(accsp.at[pl.ds(s * cp_pt, cp_pt)],
                        out.at[pl.ds(p * CH + s * cp_pt, cp_pt)])
        plsc.subcore_barrier()

        if p < P - 1:
            lax.fori_loop(0, ZR_PT, zrow, 0)


BR = 1024
NB = NP // BR


def _ka_body(x_ref, w_ref, d0_ref, d1_ref, ys_ref, dis_ref):
    deg = d0_ref[...] + d1_ref[...] + 1.0
    dis = lax.rsqrt(deg)
    y = jnp.dot(x_ref[...], w_ref[...], preferred_element_type=jnp.float32)
    ys_ref[...] = y * dis[:, None]
    dis_ref[...] = dis


def _ka_call(xp, W1, d0, d1):
    return pl.pallas_call(
        _ka_body,
        grid=(NB,),
        in_specs=[
            pl.BlockSpec((BR, D), lambda i: (i, 0)),
            pl.BlockSpec((D, D), lambda i: (0, 0)),
            pl.BlockSpec((BR,), lambda i: (i,)),
            pl.BlockSpec((BR,), lambda i: (i,)),
        ],
        out_specs=[
            pl.BlockSpec((BR, D), lambda i: (i, 0)),
            pl.BlockSpec((BR,), lambda i: (i,)),
        ],
        out_shape=[
            jax.ShapeDtypeStruct((NP, D), jnp.float32),
            jax.ShapeDtypeStruct((NP,), jnp.float32),
        ],
    )(xp, W1, d0, d1)


def _kb_body(a_ref, ys_ref, dis_ref, b_ref, w_ref, ysn_ref):
    dis = dis_ref[...]
    h = (a_ref[...] + ys_ref[...]) * dis[:, None] + b_ref[...][None, :]
    h = jnp.maximum(h, 0.0)
    y2 = jnp.dot(h, w_ref[...], preferred_element_type=jnp.float32)
    ysn_ref[...] = y2 * dis[:, None]


def _kb_call(a, ys, dis, b_t, W_t):
    return pl.pallas_call(
        _kb_body,
        grid=(NB,),
        in_specs=[
            pl.BlockSpec((BR, D), lambda i: (i, 0)),
            pl.BlockSpec((BR, D), lambda i: (i, 0)),
            pl.BlockSpec((BR,), lambda i: (i,)),
            pl.BlockSpec((D,), lambda i: (0,)),
            pl.BlockSpec((D, D), lambda i: (0, 0)),
        ],
        out_specs=pl.BlockSpec((BR, D), lambda i: (i, 0)),
        out_shape=jax.ShapeDtypeStruct((NP, D), jnp.float32),
    )(a, ys, dis, b_t, W_t)


def _kc_body(a_ref, ys2_ref, dis_ref, b_ref, bat_ref, wfc_ref,
             bfc_ref, out_ref, sums, cnt):
    i = pl.program_id(0)

    @pl.when(i == 0)
    def _():
        sums[...] = jnp.zeros_like(sums)
        cnt[...] = jnp.zeros_like(cnt)

    dis = dis_ref[...]
    h = (a_ref[...] + ys2_ref[...]) * dis[:, None] + b_ref[...][None, :]
    h = jnp.maximum(h, 0.0)
    bb = bat_ref[...]
    gid = lax.broadcasted_iota(jnp.int32, (BR, G), 1).astype(jnp.float32)
    oh = (bb[:, None] == gid).astype(jnp.float32)
    dn = (((0,), (0,)), ((), ()))
    sums[...] += lax.dot_general(oh, h, dn,
                                 preferred_element_type=jnp.float32)
    cnt[...] += lax.dot_general(oh, jnp.ones((BR, D), jnp.float32), dn,
                                preferred_element_type=jnp.float32)

    @pl.when(i == pl.num_programs(0) - 1)
    def _():
        pooled = sums[...] / jnp.maximum(cnt[...], 1.0)
        o = jnp.dot(pooled, wfc_ref[...], preferred_element_type=jnp.float32)
        out_ref[...] = jnp.maximum(o + bfc_ref[...][None, :], 0.0)


def _kc_call(a2, ys2, dis, b2, batf, Wfc, bfc):
    return pl.pallas_call(
        _kc_body,
        grid=(NB,),
        in_specs=[
            pl.BlockSpec((BR, D), lambda i: (i, 0)),
            pl.BlockSpec((BR, D), lambda i: (i, 0)),
            pl.BlockSpec((BR,), lambda i: (i,)),
            pl.BlockSpec((D,), lambda i: (0,)),
            pl.BlockSpec((BR,), lambda i: (i,)),
            pl.BlockSpec((D, D), lambda i: (0, 0)),
            pl.BlockSpec((D,), lambda i: (0,)),
        ],
        out_specs=pl.BlockSpec((G, D), lambda i: (0, 0)),
        out_shape=jax.ShapeDtypeStruct((G, D), jnp.float32),
        scratch_shapes=[
            pltpu.VMEM((G, D), jnp.float32),
            pltpu.VMEM((G, D), jnp.float32),
        ],
    )(a2, ys2, dis, b2, batf, Wfc, bfc)


def kernel(x, edge_index, batch, W1, b1, W2, b2, Wfc, bfc):
    src = edge_index[0].astype(jnp.int32)
    dst = edge_index[1].astype(jnp.int32)
    npad = EP - E
    pad_idx = N + (jnp.arange(npad, dtype=jnp.int32) % (NP - N))
    src2 = jnp.concatenate([src, pad_idx]).reshape(NGR, 128)
    dst2 = jnp.concatenate([dst, pad_idx]).reshape(NGR, 128)
    xp = jnp.pad(x, ((0, NP - N), (0, 0)))
    batf = jnp.pad(batch.astype(jnp.float32), (0, NP - N),
                   constant_values=float(G))

    d0, d1 = _deg_kernel(dst2)
    ys1, dis = _ka_call(xp, W1, d0, d1)
    a1 = _agg_chunk(ys1, src2, dst2)
    ys2 = _kb_call(a1, ys1, dis, b1, W2)
    a2 = _agg_chunk(ys2, src2, dst2)
    return _kc_call(a2, ys2, dis, b2, batf, Wfc, bfc)


# speedup vs baseline: 9.4013x; 1.9890x over previous
import functools

import jax
import jax.numpy as jnp
from jax import lax
from jax.experimental import pallas as pl
from jax.experimental.pallas import tpu as pltpu
from jax.experimental.pallas import tpu_sc as plsc

N = 10000
NP = 10240
P = 4
CH = 2816
TRASH = 128
NACC = CH + TRASH
E = 320000
D = 128
G = 64
NS = 16
NT = 2 * NS
EP = 327680
NGR = EP // 128
GPT = NGR // NT
BCAP = 3328
BGR = BCAP // 128
BPT = P * BCAP
ROWS_PT = NP // NS
ZR_PT = NACC // NS

_mesh2 = plsc.VectorSubcoreMesh(core_axis_name="c", subcore_axis_name="s")
_mesh1 = plsc.VectorSubcoreMesh(core_axis_name="c", subcore_axis_name="s",
                                num_cores=1)


@functools.partial(
    pl.kernel,
    out_type=(jax.ShapeDtypeStruct((NP,), jnp.float32),
              jax.ShapeDtypeStruct((NP,), jnp.float32),
              jax.ShapeDtypeStruct((NT * BPT,), jnp.int32),
              jax.ShapeDtypeStruct((NT * BPT,), jnp.int32)),
    mesh=_mesh2,
    compiler_params=pltpu.CompilerParams(needs_layout_passes=False),
    scratch_types=[
        pltpu.VMEM((GPT, 128), jnp.int32),
        pltpu.VMEM((GPT, 128), jnp.int32),
        pltpu.VMEM((BPT,), jnp.int32),
        pltpu.VMEM((BPT,), jnp.int32),
        pltpu.VMEM((128,), jnp.float32),
        pltpu.VMEM((ROWS_PT,), jnp.float32),
        pltpu.VMEM_SHARED((NP,), jnp.float32),
    ],
)
def _deg_part_kernel(src2_hbm, dst2_hbm, out0, out1, psrc, pdst,
                     srcb, dstb, bsrc, bdst, onesb, zbuf, degsp):
    c = lax.axis_index("c")
    s = lax.axis_index("s")
    wid = c * NS + s
    z16 = jnp.zeros((16,), jnp.float32)
    o16 = jnp.ones((16,), jnp.float32)
    i16 = lax.iota(jnp.int32, 16)
    for k in range(128 // 16):
        onesb[pl.ds(k * 16, 16)] = o16
    for k in range(ROWS_PT // 16):
        zbuf[pl.ds(k * 16, 16)] = z16
    pltpu.sync_copy(zbuf, degsp.at[pl.ds(s * ROWS_PT, ROWS_PT)])

    pltpu.sync_copy(src2_hbm.at[pl.ds(wid * GPT, GPT)], srcb)
    pltpu.sync_copy(dst2_hbm.at[pl.ds(wid * GPT, GPT)], dstb)

    def fill(i, carry):
        pat = (i16 + i * 16) & (TRASH - 1)
        bsrc[pl.ds(i * 16, 16)] = N + pat
        bdst[pl.ds(i * 16, 16)] = CH + pat
        return carry
    lax.fori_loop(0, BPT // 16, fill, 0)

    plsc.subcore_barrier()
    for g in range(GPT):
        pltpu.sync_copy(onesb, degsp.at[dstb.at[g]], add=True)

    def part(i, offs):
        g = i >> 3
        k = i & 7
        dv = dstb[g, pl.ds(k * 16, 16)]
        sv = srcb[g, pl.ds(k * 16, 16)]
        new_offs = []
        for q in range(P):
            inq = (dv >= q * CH) & (dv < (q + 1) * CH)
            cs = plsc.cumsum(inq.astype(jnp.int32))
            pos = q * BCAP + offs[q] + cs - 1
            ok = inq & (pos < (q + 1) * BCAP)
            plsc.store_scatter(bsrc, [pos], sv, mask=ok)
            plsc.store_scatter(bdst, [pos], dv - q * CH, mask=ok)
            new_offs.append(offs[q] + cs[15])
        return tuple(new_offs)
    zero = jnp.zeros((), jnp.int32)
    lax.fori_loop(0, GPT * 8, part, (zero, zero, zero, zero))

    pltpu.sync_copy(bsrc, psrc.at[pl.ds(wid * BPT, BPT)])
    pltpu.sync_copy(bdst, pdst.at[pl.ds(wid * BPT, BPT)])

    plsc.subcore_barrier()

    @pl.when(c == 0)
    def _():
        pltpu.sync_copy(degsp.at[pl.ds(s * ROWS_PT, ROWS_PT)],
                        out0.at[pl.ds(s * ROWS_PT, ROWS_PT)])

    @pl.when(c == 1)
    def _():
        pltpu.sync_copy(degsp.at[pl.ds(s * ROWS_PT, ROWS_PT)],
                        out1.at[pl.ds(s * ROWS_PT, ROWS_PT)])


@functools.partial(
    pl.kernel,
    out_type=jax.ShapeDtypeStruct((NP, D), jnp.float32),
    mesh=_mesh1,
    scratch_types=[
        pltpu.VMEM((2, 128), jnp.int32),
        pltpu.VMEM((2, 128), jnp.int32),
        pltpu.VMEM((2, 128, D), jnp.float32),
        pltpu.VMEM((ZR_PT, D), jnp.float32),
        pltpu.VMEM_SHARED((NACC, D), jnp.float32),
        pltpu.SemaphoreType.DMA,
    ],
)
def _agg_kernel(ys_hbm, psrc_hbm, pdst_hbm, out,
                srcw, dstw, rows, zrows, accsp, sem):
    s = lax.axis_index("s")
    z16 = jnp.zeros((16,), jnp.float32)

    def zrow(i, carry):
        for k in range(D // 16):
            zrows[i, pl.ds(k * 16, 16)] = z16
        return carry
    lax.fori_loop(0, ZR_PT, zrow, 0)

    for p in range(P):
        pltpu.sync_copy(zrows, accsp.at[pl.ds(s * ZR_PT, ZR_PT)])
        plsc.subcore_barrier()

        for j in range(2):
            base = (2 * s + j) * BPT + p * BCAP

            def ld(g, buf):
                pltpu.sync_copy(psrc_hbm.at[pl.ds(base + g * 128, 128)],
                                srcw.at[buf])
                pltpu.sync_copy(pdst_hbm.at[pl.ds(base + g * 128, 128)],
                                dstw.at[buf])
                return pltpu.async_copy(ys_hbm.at[srcw.at[buf]],
                                        rows.at[buf], sem)

            ld(0, 0)

            def grp(g, carry):
                buf = g & 1
                pltpu.make_async_copy(ys_hbm.at[srcw.at[buf]],
                                      rows.at[buf], sem).wait()

                @pl.when(g + 1 < BGR)
                def _():
                    ld(g + 1, 1 - buf)

                pltpu.sync_copy(rows.at[buf], accsp.at[dstw.at[buf]],
                                add=True)
                return carry
            lax.fori_loop(0, BGR, grp, 0)

        plsc.subcore_barrier()
        cp_pt = (CH if p < P - 1 else NP - (P - 1) * CH) // NS
        pltpu.sync_copy(accsp.at[pl.ds(s * cp_pt, cp_pt)],
                        out.at[pl.ds(p * CH + s * cp_pt, cp_pt)])
        plsc.subcore_barrier()


BR = 1024
NB = NP // BR


def _ka_body(x_ref, w_ref, d0_ref, d1_ref, ys_ref, dis_ref):
    deg = d0_ref[...] + d1_ref[...] + 1.0
    dis = lax.rsqrt(deg)
    y = jnp.dot(x_ref[...], w_ref[...], preferred_element_type=jnp.float32)
    ys_ref[...] = y * dis[:, None]
    dis_ref[...] = dis


def _ka_call(xp, W1, d0, d1):
    return pl.pallas_call(
        _ka_body,
        grid=(NB,),
        in_specs=[
            pl.BlockSpec((BR, D), lambda i: (i, 0)),
            pl.BlockSpec((D, D), lambda i: (0, 0)),
            pl.BlockSpec((BR,), lambda i: (i,)),
            pl.BlockSpec((BR,), lambda i: (i,)),
        ],
        out_specs=[
            pl.BlockSpec((BR, D), lambda i: (i, 0)),
            pl.BlockSpec((BR,), lambda i: (i,)),
        ],
        out_shape=[
            jax.ShapeDtypeStruct((NP, D), jnp.float32),
            jax.ShapeDtypeStruct((NP,), jnp.float32),
        ],
    )(xp, W1, d0, d1)


def _kb_body(a_ref, ys_ref, dis_ref, b_ref, w_ref, ysn_ref):
    dis = dis_ref[...]
    h = (a_ref[...] + ys_ref[...]) * dis[:, None] + b_ref[...][None, :]
    h = jnp.maximum(h, 0.0)
    y2 = jnp.dot(h, w_ref[...], preferred_element_type=jnp.float32)
    ysn_ref[...] = y2 * dis[:, None]


def _kb_call(a, ys, dis, b_t, W_t):
    return pl.pallas_call(
        _kb_body,
        grid=(NB,),
        in_specs=[
            pl.BlockSpec((BR, D), lambda i: (i, 0)),
            pl.BlockSpec((BR, D), lambda i: (i, 0)),
            pl.BlockSpec((BR,), lambda i: (i,)),
            pl.BlockSpec((D,), lambda i: (0,)),
            pl.BlockSpec((D, D), lambda i: (0, 0)),
        ],
        out_specs=pl.BlockSpec((BR, D), lambda i: (i, 0)),
        out_shape=jax.ShapeDtypeStruct((NP, D), jnp.float32),
    )(a, ys, dis, b_t, W_t)


def _kc_body(a_ref, ys2_ref, dis_ref, b_ref, bat_ref, wfc_ref,
             bfc_ref, out_ref, sums, cnt):
    i = pl.program_id(0)

    @pl.when(i == 0)
    def _():
        sums[...] = jnp.zeros_like(sums)
        cnt[...] = jnp.zeros_like(cnt)

    dis = dis_ref[...]
    h = (a_ref[...] + ys2_ref[...]) * dis[:, None] + b_ref[...][None, :]
    h = jnp.maximum(h, 0.0)
    bb = bat_ref[...]
    gid = lax.broadcasted_iota(jnp.int32, (BR, G), 1).astype(jnp.float32)
    oh = (bb[:, None] == gid).astype(jnp.float32)
    dn = (((0,), (0,)), ((), ()))
    sums[...] += lax.dot_general(oh, h, dn,
                                 preferred_element_type=jnp.float32)
    cnt[...] += lax.dot_general(oh, jnp.ones((BR, D), jnp.float32), dn,
                                preferred_element_type=jnp.float32)

    @pl.when(i == pl.num_programs(0) - 1)
    def _():
        pooled = sums[...] / jnp.maximum(cnt[...], 1.0)
        o = jnp.dot(pooled, wfc_ref[...], preferred_element_type=jnp.float32)
        out_ref[...] = jnp.maximum(o + bfc_ref[...][None, :], 0.0)


def _kc_call(a2, ys2, dis, b2, batf, Wfc, bfc):
    return pl.pallas_call(
        _kc_body,
        grid=(NB,),
        in_specs=[
            pl.BlockSpec((BR, D), lambda i: (i, 0)),
            pl.BlockSpec((BR, D), lambda i: (i, 0)),
            pl.BlockSpec((BR,), lambda i: (i,)),
            pl.BlockSpec((D,), lambda i: (0,)),
            pl.BlockSpec((BR,), lambda i: (i,)),
            pl.BlockSpec((D, D), lambda i: (0, 0)),
            pl.BlockSpec((D,), lambda i: (0,)),
        ],
        out_specs=pl.BlockSpec((G, D), lambda i: (0, 0)),
        out_shape=jax.ShapeDtypeStruct((G, D), jnp.float32),
        scratch_shapes=[
            pltpu.VMEM((G, D), jnp.float32),
            pltpu.VMEM((G, D), jnp.float32),
        ],
    )(a2, ys2, dis, b2, batf, Wfc, bfc)


def kernel(x, edge_index, batch, W1, b1, W2, b2, Wfc, bfc):
    src = edge_index[0].astype(jnp.int32)
    dst = edge_index[1].astype(jnp.int32)
    npad = EP - E
    pad_idx = N + (jnp.arange(npad, dtype=jnp.int32) % (NP - N))
    src2 = jnp.concatenate([src, pad_idx]).reshape(NGR, 128)
    dst2 = jnp.concatenate([dst, pad_idx]).reshape(NGR, 128)
    xp = jnp.pad(x, ((0, NP - N), (0, 0)))
    batf = jnp.pad(batch.astype(jnp.float32), (0, NP - N),
                   constant_values=float(G))

    d0, d1, psrc, pdst = _deg_part_kernel(src2, dst2)
    ys1, dis = _ka_call(xp, W1, d0, d1)
    a1 = _agg_kernel(ys1, psrc, pdst)
    ys2 = _kb_call(a1, ys1, dis, b1, W2)
    a2 = _agg_kernel(ys2, psrc, pdst)
    return _kc_call(a2, ys2, dis, b2, batf, Wfc, bfc)


# speedup vs baseline: 14.6634x; 1.5597x over previous
import functools

import jax
import jax.numpy as jnp
from jax import lax
from jax.experimental import pallas as pl
from jax.experimental.pallas import tpu as pltpu
from jax.experimental.pallas import tpu_sc as plsc

N = 10000
NP = 10240
P = 4
CH = 2816
TRASH = 128
NACC = CH + TRASH
E = 320000
D = 128
G = 64
NS = 16
NT = 2 * NS
EP = 327680
NGR = EP // 128
GPT = NGR // NT
BCAP = 4096
BGR = BCAP // 128
BPT = P * BCAP
ROWS_PT = NP // NS
ZR_PT = NACC // NS

_mesh2 = plsc.VectorSubcoreMesh(core_axis_name="c", subcore_axis_name="s")
_mesh1 = plsc.VectorSubcoreMesh(core_axis_name="c", subcore_axis_name="s",
                                num_cores=1)


@functools.partial(
    pl.kernel,
    out_type=(jax.ShapeDtypeStruct((NP,), jnp.float32),
              jax.ShapeDtypeStruct((NP,), jnp.float32),
              jax.ShapeDtypeStruct((NT * BPT // 128, 128), jnp.int32),
              jax.ShapeDtypeStruct((NT * BPT // 128, 128), jnp.int32)),
    mesh=_mesh2,
    compiler_params=pltpu.CompilerParams(needs_layout_passes=False),
    scratch_types=[
        pltpu.VMEM((GPT, 128), jnp.int32),
        pltpu.VMEM((GPT, 128), jnp.int32),
        pltpu.VMEM((BPT // 128, 128), jnp.int32),
        pltpu.VMEM((BPT // 128, 128), jnp.int32),
        pltpu.VMEM((128,), jnp.float32),
        pltpu.VMEM((ROWS_PT,), jnp.float32),
        pltpu.VMEM_SHARED((NP,), jnp.float32),
    ],
)
def _deg_part_kernel(src2_hbm, dst2_hbm, out0, out1, psrc, pdst,
                     srcb, dstb, bsrc, bdst, onesb, zbuf, degsp):
    c = lax.axis_index("c")
    s = lax.axis_index("s")
    wid = c * NS + s
    z16 = jnp.zeros((16,), jnp.float32)
    o16 = jnp.ones((16,), jnp.float32)
    i16 = lax.iota(jnp.int32, 16)
    for k in range(128 // 16):
        onesb[pl.ds(k * 16, 16)] = o16
    for k in range(ROWS_PT // 16):
        zbuf[pl.ds(k * 16, 16)] = z16
    pltpu.sync_copy(zbuf, degsp.at[pl.ds(s * ROWS_PT, ROWS_PT)])

    pltpu.sync_copy(src2_hbm.at[pl.ds(wid * GPT, GPT)], srcb)
    pltpu.sync_copy(dst2_hbm.at[pl.ds(wid * GPT, GPT)], dstb)

    def fill(i, carry):
        r = i >> 3
        k = i & 7
        pat = (i16 + i * 16) & (TRASH - 1)
        bsrc[r, pl.ds(k * 16, 16)] = N + pat
        bdst[r, pl.ds(k * 16, 16)] = CH + pat
        return carry
    lax.fori_loop(0, BPT // 16, fill, 0)

    plsc.subcore_barrier()
    for g in range(GPT):
        pltpu.sync_copy(onesb, degsp.at[dstb.at[g]], add=True)

    def part(i, offs):
        g = i >> 3
        k = i & 7
        dv = dstb[g, pl.ds(k * 16, 16)]
        sv = srcb[g, pl.ds(k * 16, 16)]
        new_offs = []
        for q in range(P):
            inq = (dv >= q * CH) & (dv < (q + 1) * CH)
            cs = plsc.cumsum(inq.astype(jnp.int32))
            pos = q * BCAP + offs[q] + cs - 1
            ok = inq & (pos < (q + 1) * BCAP)
            plsc.store_scatter(bsrc, [pos >> 7, pos & 127], sv, mask=ok)
            plsc.store_scatter(bdst, [pos >> 7, pos & 127], dv - q * CH,
                               mask=ok)
            new_offs.append(offs[q] + cs[15])
        return tuple(new_offs)
    zero = jnp.zeros((), jnp.int32)
    lax.fori_loop(0, GPT * 8, part, (zero, zero, zero, zero))

    gpr = BPT // 128
    pltpu.sync_copy(bsrc, psrc.at[pl.ds(wid * gpr, gpr)])
    pltpu.sync_copy(bdst, pdst.at[pl.ds(wid * gpr, gpr)])

    plsc.subcore_barrier()

    @pl.when(c == 0)
    def _():
        pltpu.sync_copy(degsp.at[pl.ds(s * ROWS_PT, ROWS_PT)],
                        out0.at[pl.ds(s * ROWS_PT, ROWS_PT)])

    @pl.when(c == 1)
    def _():
        pltpu.sync_copy(degsp.at[pl.ds(s * ROWS_PT, ROWS_PT)],
                        out1.at[pl.ds(s * ROWS_PT, ROWS_PT)])


@functools.partial(
    pl.kernel,
    out_type=jax.ShapeDtypeStruct((NP, D), jnp.float32),
    mesh=_mesh1,
    scratch_types=[
        pltpu.VMEM((BGR, 128), jnp.int32),
        pltpu.VMEM((BGR, 128), jnp.int32),
        pltpu.VMEM((4, 128, D), jnp.float32),
        pltpu.VMEM((ZR_PT, D), jnp.float32),
        pltpu.VMEM_SHARED((NACC, D), jnp.float32),
        pltpu.SemaphoreType.DMA,
        pltpu.SemaphoreType.DMA,
    ],
)
def _agg_kernel(ys_hbm, psrc_hbm, pdst_hbm, out,
                srcr, dstr, rows, zrows, accsp, semg, sems):
    s = lax.axis_index("s")
    z16 = jnp.zeros((16,), jnp.float32)

    def zrow(i, carry):
        for k in range(D // 16):
            zrows[i, pl.ds(k * 16, 16)] = z16
        return carry
    lax.fori_loop(0, ZR_PT, zrow, 0)

    for p in range(P):
        pltpu.sync_copy(zrows, accsp.at[pl.ds(s * ZR_PT, ZR_PT)])
        plsc.subcore_barrier()

        for j in range(2):
            gbase = (2 * s + j) * (BPT // 128) + p * BGR
            pltpu.sync_copy(psrc_hbm.at[pl.ds(gbase, BGR)], srcr)
            pltpu.sync_copy(pdst_hbm.at[pl.ds(gbase, BGR)], dstr)

            for t in range(3):
                pltpu.async_copy(ys_hbm.at[srcr.at[t]], rows.at[t], semg)

            def grp(g, carry):
                buf = g & 3
                pltpu.make_async_copy(ys_hbm.at[srcr.at[0]],
                                      rows.at[buf], semg).wait()
                pltpu.async_copy(rows.at[buf], accsp.at[dstr.at[g]], sems,
                                 add=True)

                @pl.when(g >= 1)
                def _():
                    pltpu.make_async_copy(rows.at[0], accsp.at[dstr.at[0]],
                                          sems).wait()

                @pl.when(g + 3 < BGR)
                def _():
                    pltpu.async_copy(ys_hbm.at[srcr.at[g + 3]],
                                     rows.at[(g + 3) & 3], semg)
                return carry
            lax.fori_loop(0, BGR, grp, 0)
            pltpu.make_async_copy(rows.at[0], accsp.at[dstr.at[0]],
                                  sems).wait()

        plsc.subcore_barrier()
        cp_pt = (CH if p < P - 1 else NP - (P - 1) * CH) // NS
        pltpu.sync_copy(accsp.at[pl.ds(s * cp_pt, cp_pt)],
                        out.at[pl.ds(p * CH + s * cp_pt, cp_pt)])
        plsc.subcore_barrier()


BR = 1024
NB = NP // BR


def _ka_body(x_ref, w_ref, d0_ref, d1_ref, ys_ref, dis_ref):
    deg = d0_ref[...] + d1_ref[...] + 1.0
    dis = lax.rsqrt(deg)
    y = jnp.dot(x_ref[...], w_ref[...], preferred_element_type=jnp.float32)
    ys_ref[...] = y * dis[:, None]
    dis_ref[...] = dis


def _ka_call(xp, W1, d0, d1):
    return pl.pallas_call(
        _ka_body,
        grid=(NB,),
        in_specs=[
            pl.BlockSpec((BR, D), lambda i: (i, 0)),
            pl.BlockSpec((D, D), lambda i: (0, 0)),
            pl.BlockSpec((BR,), lambda i: (i,)),
            pl.BlockSpec((BR,), lambda i: (i,)),
        ],
        out_specs=[
            pl.BlockSpec((BR, D), lambda i: (i, 0)),
            pl.BlockSpec((BR,), lambda i: (i,)),
        ],
        out_shape=[
            jax.ShapeDtypeStruct((NP, D), jnp.float32),
            jax.ShapeDtypeStruct((NP,), jnp.float32),
        ],
    )(xp, W1, d0, d1)


def _kb_body(a_ref, ys_ref, dis_ref, b_ref, w_ref, ysn_ref):
    dis = dis_ref[...]
    h = (a_ref[...] + ys_ref[...]) * dis[:, None] + b_ref[...][None, :]
    h = jnp.maximum(h, 0.0)
    y2 = jnp.dot(h, w_ref[...], preferred_element_type=jnp.float32)
    ysn_ref[...] = y2 * dis[:, None]


def _kb_call(a, ys, dis, b_t, W_t):
    return pl.pallas_call(
        _kb_body,
        grid=(NB,),
        in_specs=[
            pl.BlockSpec((BR, D), lambda i: (i, 0)),
            pl.BlockSpec((BR, D), lambda i: (i, 0)),
            pl.BlockSpec((BR,), lambda i: (i,)),
            pl.BlockSpec((D,), lambda i: (0,)),
            pl.BlockSpec((D, D), lambda i: (0, 0)),
        ],
        out_specs=pl.BlockSpec((BR, D), lambda i: (i, 0)),
        out_shape=jax.ShapeDtypeStruct((NP, D), jnp.float32),
    )(a, ys, dis, b_t, W_t)


def _kc_body(a_ref, ys2_ref, dis_ref, b_ref, bat_ref, wfc_ref,
             bfc_ref, out_ref, sums, cnt):
    i = pl.program_id(0)

    @pl.when(i == 0)
    def _():
        sums[...] = jnp.zeros_like(sums)
        cnt[...] = jnp.zeros_like(cnt)

    dis = dis_ref[...]
    h = (a_ref[...] + ys2_ref[...]) * dis[:, None] + b_ref[...][None, :]
    h = jnp.maximum(h, 0.0)
    bb = bat_ref[...]
    gid = lax.broadcasted_iota(jnp.int32, (BR, G), 1).astype(jnp.float32)
    oh = (bb[:, None] == gid).astype(jnp.float32)
    dn = (((0,), (0,)), ((), ()))
    sums[...] += lax.dot_general(oh, h, dn,
                                 preferred_element_type=jnp.float32)
    cnt[...] += lax.dot_general(oh, jnp.ones((BR, D), jnp.float32), dn,
                                preferred_element_type=jnp.float32)

    @pl.when(i == pl.num_programs(0) - 1)
    def _():
        pooled = sums[...] / jnp.maximum(cnt[...], 1.0)
        o = jnp.dot(pooled, wfc_ref[...], preferred_element_type=jnp.float32)
        out_ref[...] = jnp.maximum(o + bfc_ref[...][None, :], 0.0)


def _kc_call(a2, ys2, dis, b2, batf, Wfc, bfc):
    return pl.pallas_call(
        _kc_body,
        grid=(NB,),
        in_specs=[
            pl.BlockSpec((BR, D), lambda i: (i, 0)),
            pl.BlockSpec((BR, D), lambda i: (i, 0)),
            pl.BlockSpec((BR,), lambda i: (i,)),
            pl.BlockSpec((D,), lambda i: (0,)),
            pl.BlockSpec((BR,), lambda i: (i,)),
            pl.BlockSpec((D, D), lambda i: (0, 0)),
            pl.BlockSpec((D,), lambda i: (0,)),
        ],
        out_specs=pl.BlockSpec((G, D), lambda i: (0, 0)),
        out_shape=jax.ShapeDtypeStruct((G, D), jnp.float32),
        scratch_shapes=[
            pltpu.VMEM((G, D), jnp.float32),
            pltpu.VMEM((G, D), jnp.float32),
        ],
    )(a2, ys2, dis, b2, batf, Wfc, bfc)


def kernel(x, edge_index, batch, W1, b1, W2, b2, Wfc, bfc):
    src = edge_index[0].astype(jnp.int32)
    dst = edge_index[1].astype(jnp.int32)
    npad = EP - E
    pad_idx = N + (jnp.arange(npad, dtype=jnp.int32) % (NP - N))
    src2 = jnp.concatenate([src, pad_idx]).reshape(NGR, 128)
    dst2 = jnp.concatenate([dst, pad_idx]).reshape(NGR, 128)
    xp = jnp.pad(x, ((0, NP - N), (0, 0)))
    batf = jnp.pad(batch.astype(jnp.float32), (0, NP - N),
                   constant_values=float(G))

    d0, d1, psrc, pdst = _deg_part_kernel(src2, dst2)
    ys1, dis = _ka_call(xp, W1, d0, d1)
    a1 = _agg_kernel(ys1, psrc, pdst)
    ys2 = _kb_call(a1, ys1, dis, b1, W2)
    a2 = _agg_kernel(ys2, psrc, pdst)
    return _kc_call(a2, ys2, dis, b2, batf, Wfc, bfc)


# speedup vs baseline: 19.5798x; 1.3353x over previous
import functools

import jax
import jax.numpy as jnp
from jax import lax
from jax.experimental import pallas as pl
from jax.experimental.pallas import tpu as pltpu
from jax.experimental.pallas import tpu_sc as plsc

N = 10000
NP = 10240
P = 4
CH = 2560
TRASH = 128
NACC = CH + TRASH
E = 320000
D = 128
G = 64
NS = 16
NT = 2 * NS
EP = 327680
NGR = EP // 128
GPT = NGR // NT
BCAP = 4096
BGR = BCAP // 128
BPT = P * BCAP
ROWS_PT = NP // NS
ZR_PT = NACC // NS
CP_PT = CH // NS

_mesh2 = plsc.VectorSubcoreMesh(core_axis_name="c", subcore_axis_name="s")
_mesh1 = plsc.VectorSubcoreMesh(core_axis_name="c", subcore_axis_name="s",
                                num_cores=1)


@functools.partial(
    pl.kernel,
    out_type=(jax.ShapeDtypeStruct((NP,), jnp.float32),
              jax.ShapeDtypeStruct((NP,), jnp.float32),
              jax.ShapeDtypeStruct((NT * BPT // 128, 128), jnp.int32),
              jax.ShapeDtypeStruct((NT * BPT // 128, 128), jnp.int32)),
    mesh=_mesh2,
    compiler_params=pltpu.CompilerParams(needs_layout_passes=False),
    scratch_types=[
        pltpu.VMEM((GPT, 128), jnp.int32),
        pltpu.VMEM((GPT, 128), jnp.int32),
        pltpu.VMEM((BPT // 128, 128), jnp.int32),
        pltpu.VMEM((BPT // 128, 128), jnp.int32),
        pltpu.VMEM((128,), jnp.float32),
        pltpu.VMEM((ROWS_PT,), jnp.float32),
        pltpu.VMEM_SHARED((NP,), jnp.float32),
    ],
)
def _deg_part_kernel(src2_hbm, dst2_hbm, out0, out1, psrc, pdst,
                     srcb, dstb, bsrc, bdst, onesb, zbuf, degsp):
    c = lax.axis_index("c")
    s = lax.axis_index("s")
    wid = c * NS + s
    z16 = jnp.zeros((16,), jnp.float32)
    o16 = jnp.ones((16,), jnp.float32)
    i16 = lax.iota(jnp.int32, 16)
    for k in range(128 // 16):
        onesb[pl.ds(k * 16, 16)] = o16
    for k in range(ROWS_PT // 16):
        zbuf[pl.ds(k * 16, 16)] = z16
    pltpu.sync_copy(zbuf, degsp.at[pl.ds(s * ROWS_PT, ROWS_PT)])

    pltpu.sync_copy(src2_hbm.at[pl.ds(wid * GPT, GPT)], srcb)
    pltpu.sync_copy(dst2_hbm.at[pl.ds(wid * GPT, GPT)], dstb)

    def fill(i, carry):
        r = i >> 3
        k = i & 7
        pat = (i16 + i * 16) & (TRASH - 1)
        bsrc[r, pl.ds(k * 16, 16)] = N + pat
        bdst[r, pl.ds(k * 16, 16)] = CH + pat
        return carry
    lax.fori_loop(0, BPT // 16, fill, 0)

    plsc.subcore_barrier()
    for g in range(GPT):
        pltpu.sync_copy(onesb, degsp.at[dstb.at[g]], add=True)

    def part(i, offs):
        g = i >> 3
        k = i & 7
        dv = dstb[g, pl.ds(k * 16, 16)]
        sv = srcb[g, pl.ds(k * 16, 16)]
        new_offs = []
        for q in range(P):
            inq = (dv >= q * CH) & (dv < (q + 1) * CH)
            cs = plsc.cumsum(inq.astype(jnp.int32))
            pos = q * BCAP + offs[q] + cs - 1
            ok = inq & (pos < (q + 1) * BCAP)
            plsc.store_scatter(bsrc, [pos >> 7, pos & 127], sv, mask=ok)
            plsc.store_scatter(bdst, [pos >> 7, pos & 127], dv - q * CH,
                               mask=ok)
            new_offs.append(offs[q] + cs[15])
        return tuple(new_offs)
    zero = jnp.zeros((), jnp.int32)
    lax.fori_loop(0, GPT * 8, part, (zero, zero, zero, zero))

    gpr = BPT // 128
    pltpu.sync_copy(bsrc, psrc.at[pl.ds(wid * gpr, gpr)])
    pltpu.sync_copy(bdst, pdst.at[pl.ds(wid * gpr, gpr)])

    plsc.subcore_barrier()

    @pl.when(c == 0)
    def _():
        pltpu.sync_copy(degsp.at[pl.ds(s * ROWS_PT, ROWS_PT)],
                        out0.at[pl.ds(s * ROWS_PT, ROWS_PT)])

    @pl.when(c == 1)
    def _():
        pltpu.sync_copy(degsp.at[pl.ds(s * ROWS_PT, ROWS_PT)],
                        out1.at[pl.ds(s * ROWS_PT, ROWS_PT)])


@functools.partial(
    pl.kernel,
    out_type=jax.ShapeDtypeStruct((NP, D), jnp.float32),
    mesh=_mesh2,
    scratch_types=[
        pltpu.VMEM((BGR, 128), jnp.int32),
        pltpu.VMEM((BGR, 128), jnp.int32),
        pltpu.VMEM((4, 128, D), jnp.float32),
        pltpu.VMEM((ZR_PT, D), jnp.float32),
        pltpu.VMEM_SHARED((NACC, D), jnp.float32),
        pltpu.SemaphoreType.DMA,
        pltpu.SemaphoreType.DMA,
    ],
)
def _agg_kernel(ys_hbm, psrc_hbm, pdst_hbm, out,
                srcr, dstr, rows, zrows, accsp, semg, sems):
    c = lax.axis_index("c")
    s = lax.axis_index("s")
    z16 = jnp.zeros((16,), jnp.float32)

    def zrow(i, carry):
        for k in range(D // 16):
            zrows[i, pl.ds(k * 16, 16)] = z16
        return carry
    lax.fori_loop(0, ZR_PT, zrow, 0)

    for p in range(P // 2):
        q = 2 * c + p
        pltpu.sync_copy(zrows, accsp.at[pl.ds(s * ZR_PT, ZR_PT)])
        plsc.subcore_barrier()

        for j in range(2):
            gbase = (2 * s + j) * (BPT // 128) + q * BGR
            pltpu.sync_copy(psrc_hbm.at[pl.ds(gbase, BGR)], srcr)
            pltpu.sync_copy(pdst_hbm.at[pl.ds(gbase, BGR)], dstr)

            for t in range(3):
                pltpu.async_copy(ys_hbm.at[srcr.at[t]], rows.at[t], semg)

            def grp(g, carry):
                buf = g & 3
                pltpu.make_async_copy(ys_hbm.at[srcr.at[0]],
                                      rows.at[buf], semg).wait()
                pltpu.async_copy(rows.at[buf], accsp.at[dstr.at[g]], sems,
                                 add=True)

                @pl.when(g >= 1)
                def _():
                    pltpu.make_async_copy(rows.at[0], accsp.at[dstr.at[0]],
                                          sems).wait()

                @pl.when(g + 3 < BGR)
                def _():
                    pltpu.async_copy(ys_hbm.at[srcr.at[g + 3]],
                                     rows.at[(g + 3) & 3], semg)
                return carry
            lax.fori_loop(0, BGR, grp, 0)
            pltpu.make_async_copy(rows.at[0], accsp.at[dstr.at[0]],
                                  sems).wait()

        plsc.subcore_barrier()
        pltpu.sync_copy(accsp.at[pl.ds(s * CP_PT, CP_PT)],
                        out.at[pl.ds(q * CH + s * CP_PT, CP_PT)])
        plsc.subcore_barrier()


BR = 1024
NB = NP // BR


def _ka_body(x_ref, w_ref, d0_ref, d1_ref, ys_ref, dis_ref):
    deg = d0_ref[...] + d1_ref[...] + 1.0
    dis = lax.rsqrt(deg)
    y = jnp.dot(x_ref[...], w_ref[...], preferred_element_type=jnp.float32)
    ys_ref[...] = y * dis[:, None]
    dis_ref[...] = dis


def _ka_call(xp, W1, d0, d1):
    return pl.pallas_call(
        _ka_body,
        grid=(NB,),
        in_specs=[
            pl.BlockSpec((BR, D), lambda i: (i, 0)),
            pl.BlockSpec((D, D), lambda i: (0, 0)),
            pl.BlockSpec((BR,), lambda i: (i,)),
            pl.BlockSpec((BR,), lambda i: (i,)),
        ],
        out_specs=[
            pl.BlockSpec((BR, D), lambda i: (i, 0)),
            pl.BlockSpec((BR,), lambda i: (i,)),
        ],
        out_shape=[
            jax.ShapeDtypeStruct((NP, D), jnp.float32),
            jax.ShapeDtypeStruct((NP,), jnp.float32),
        ],
    )(xp, W1, d0, d1)


def _kb_body(a_ref, ys_ref, dis_ref, b_ref, w_ref, ysn_ref):
    dis = dis_ref[...]
    h = (a_ref[...] + ys_ref[...]) * dis[:, None] + b_ref[...][None, :]
    h = jnp.maximum(h, 0.0)
    y2 = jnp.dot(h, w_ref[...], preferred_element_type=jnp.float32)
    ysn_ref[...] = y2 * dis[:, None]


def _kb_call(a, ys, dis, b_t, W_t):
    return pl.pallas_call(
        _kb_body,
        grid=(NB,),
        in_specs=[
            pl.BlockSpec((BR, D), lambda i: (i, 0)),
            pl.BlockSpec((BR, D), lambda i: (i, 0)),
            pl.BlockSpec((BR,), lambda i: (i,)),
            pl.BlockSpec((D,), lambda i: (0,)),
            pl.BlockSpec((D, D), lambda i: (0, 0)),
        ],
        out_specs=pl.BlockSpec((BR, D), lambda i: (i, 0)),
        out_shape=jax.ShapeDtypeStruct((NP, D), jnp.float32),
    )(a, ys, dis, b_t, W_t)


def _kc_body(a_ref, ys2_ref, dis_ref, b_ref, bat_ref, wfc_ref,
             bfc_ref, out_ref, sums, cnt):
    i = pl.program_id(0)

    @pl.when(i == 0)
    def _():
        sums[...] = jnp.zeros_like(sums)
        cnt[...] = jnp.zeros_like(cnt)

    dis = dis_ref[...]
    h = (a_ref[...] + ys2_ref[...]) * dis[:, None] + b_ref[...][None, :]
    h = jnp.maximum(h, 0.0)
    bb = bat_ref[...]
    gid = lax.broadcasted_iota(jnp.int32, (BR, G), 1).astype(jnp.float32)
    oh = (bb[:, None] == gid).astype(jnp.float32)
    dn = (((0,), (0,)), ((), ()))
    sums[...] += lax.dot_general(oh, h, dn,
                                 preferred_element_type=jnp.float32)
    cnt[...] += lax.dot_general(oh, jnp.ones((BR, D), jnp.float32), dn,
                                preferred_element_type=jnp.float32)

    @pl.when(i == pl.num_programs(0) - 1)
    def _():
        pooled = sums[...] / jnp.maximum(cnt[...], 1.0)
        o = jnp.dot(pooled, wfc_ref[...], preferred_element_type=jnp.float32)
        out_ref[...] = jnp.maximum(o + bfc_ref[...][None, :], 0.0)


def _kc_call(a2, ys2, dis, b2, batf, Wfc, bfc):
    return pl.pallas_call(
        _kc_body,
        grid=(NB,),
        in_specs=[
            pl.BlockSpec((BR, D), lambda i: (i, 0)),
            pl.BlockSpec((BR, D), lambda i: (i, 0)),
            pl.BlockSpec((BR,), lambda i: (i,)),
            pl.BlockSpec((D,), lambda i: (0,)),
            pl.BlockSpec((BR,), lambda i: (i,)),
            pl.BlockSpec((D, D), lambda i: (0, 0)),
            pl.BlockSpec((D,), lambda i: (0,)),
        ],
        out_specs=pl.BlockSpec((G, D), lambda i: (0, 0)),
        out_shape=jax.ShapeDtypeStruct((G, D), jnp.float32),
        scratch_shapes=[
            pltpu.VMEM((G, D), jnp.float32),
            pltpu.VMEM((G, D), jnp.float32),
        ],
    )(a2, ys2, dis, b2, batf, Wfc, bfc)


def kernel(x, edge_index, batch, W1, b1, W2, b2, Wfc, bfc):
    src = edge_index[0].astype(jnp.int32)
    dst = edge_index[1].astype(jnp.int32)
    npad = EP - E
    pad_idx = N + (jnp.arange(npad, dtype=jnp.int32) % (NP - N))
    src2 = jnp.concatenate([src, pad_idx]).reshape(NGR, 128)
    dst2 = jnp.concatenate([dst, pad_idx]).reshape(NGR, 128)
    xp = jnp.pad(x, ((0, NP - N), (0, 0)))
    batf = jnp.pad(batch.astype(jnp.float32), (0, NP - N),
                   constant_values=float(G))

    d0, d1, psrc, pdst = _deg_part_kernel(src2, dst2)
    ys1, dis = _ka_call(xp, W1, d0, d1)
    a1 = _agg_kernel(ys1, psrc, pdst)
    ys2 = _kb_call(a1, ys1, dis, b1, W2)
    a2 = _agg_kernel(ys2, psrc, pdst)
    return _kc_call(a2, ys2, dis, b2, batf, Wfc, bfc)


# speedup vs baseline: 33.5455x; 1.7133x over previous
import functools

import jax
import jax.numpy as jnp
from jax import lax
from jax.experimental import pallas as pl
from jax.experimental.pallas import tpu as pltpu
from jax.experimental.pallas import tpu_sc as plsc

N = 10000
NP = 10240
P = 4
CH = 2560
TRASH = 128
NACC = CH + TRASH
E = 320000
D = 128
G = 64
NS = 16
NT = 2 * NS
EP = 327680
NGR = EP // 128
GPT = NGR // NT
BCAP = 4096
BGR = BCAP // 128
BPT = P * BCAP
ROWS_PT = NP // NS
ZR_PT = NACC // NS
CP_PT = CH // NS

_mesh2 = plsc.VectorSubcoreMesh(core_axis_name="c", subcore_axis_name="s")
_mesh1 = plsc.VectorSubcoreMesh(core_axis_name="c", subcore_axis_name="s",
                                num_cores=1)


@functools.partial(
    pl.kernel,
    out_type=(jax.ShapeDtypeStruct((NP,), jnp.float32),
              jax.ShapeDtypeStruct((NP,), jnp.float32),
              jax.ShapeDtypeStruct((NT * BPT // 128, 128), jnp.int32),
              jax.ShapeDtypeStruct((NT * BPT // 128, 128), jnp.int32),
              jax.ShapeDtypeStruct((NT * 16,), jnp.int32)),
    mesh=_mesh2,
    compiler_params=pltpu.CompilerParams(needs_layout_passes=False),
    scratch_types=[
        pltpu.VMEM((GPT, 128), jnp.int32),
        pltpu.VMEM((GPT, 128), jnp.int32),
        pltpu.VMEM((BPT // 128, 128), jnp.int32),
        pltpu.VMEM((BPT // 128, 128), jnp.int32),
        pltpu.VMEM((16,), jnp.int32),
        pltpu.VMEM((128,), jnp.float32),
        pltpu.VMEM((ROWS_PT,), jnp.float32),
        pltpu.VMEM_SHARED((NP,), jnp.float32),
    ],
)
def _deg_part_kernel(src2_hbm, dst2_hbm, out0, out1, psrc, pdst, pcnt,
                     srcb, dstb, bsrc, bdst, cntb, onesb, zbuf, degsp):
    c = lax.axis_index("c")
    s = lax.axis_index("s")
    wid = c * NS + s
    z16 = jnp.zeros((16,), jnp.float32)
    o16 = jnp.ones((16,), jnp.float32)
    i16 = lax.iota(jnp.int32, 16)
    for k in range(128 // 16):
        onesb[pl.ds(k * 16, 16)] = o16
    for k in range(ROWS_PT // 16):
        zbuf[pl.ds(k * 16, 16)] = z16
    pltpu.sync_copy(zbuf, degsp.at[pl.ds(s * ROWS_PT, ROWS_PT)])

    pltpu.sync_copy(src2_hbm.at[pl.ds(wid * GPT, GPT)], srcb)
    pltpu.sync_copy(dst2_hbm.at[pl.ds(wid * GPT, GPT)], dstb)

    def fill(i, carry):
        r = i >> 3
        k = i & 7
        pat = (i16 + i * 16) & (TRASH - 1)
        bsrc[r, pl.ds(k * 16, 16)] = N + pat
        bdst[r, pl.ds(k * 16, 16)] = CH + pat
        return carry
    lax.fori_loop(0, BPT // 16, fill, 0)

    plsc.subcore_barrier()
    for g in range(GPT):
        pltpu.sync_copy(onesb, degsp.at[dstb.at[g]], add=True)

    def part(i, offs):
        g = i >> 3
        k = i & 7
        dv = dstb[g, pl.ds(k * 16, 16)]
        sv = srcb[g, pl.ds(k * 16, 16)]
        new_offs = []
        for q in range(P):
            inq = (dv >= q * CH) & (dv < (q + 1) * CH)
            cs = plsc.cumsum(inq.astype(jnp.int32))
            pos = q * BCAP + offs[q] + cs - 1
            ok = inq & (pos < (q + 1) * BCAP)
            plsc.store_scatter(bsrc, [pos >> 7, pos & 127], sv, mask=ok)
            plsc.store_scatter(bdst, [pos >> 7, pos & 127], dv - q * CH,
                               mask=ok)
            new_offs.append(offs[q] + cs[15])
        return tuple(new_offs)
    zero = jnp.zeros((), jnp.int32)
    offs = lax.fori_loop(0, GPT * 8, part, (zero, zero, zero, zero))
    cvec = jnp.zeros((16,), jnp.int32)
    for q in range(P):
        cvec = jnp.where(i16 == q, jnp.minimum(offs[q], BCAP), cvec)
    cntb[...] = cvec
    pltpu.sync_copy(cntb, pcnt.at[pl.ds(wid * 16, 16)])

    gpr = BPT // 128
    pltpu.sync_copy(bsrc, psrc.at[pl.ds(wid * gpr, gpr)])
    pltpu.sync_copy(bdst, pdst.at[pl.ds(wid * gpr, gpr)])

    plsc.subcore_barrier()

    @pl.when(c == 0)
    def _():
        pltpu.sync_copy(degsp.at[pl.ds(s * ROWS_PT, ROWS_PT)],
                        out0.at[pl.ds(s * ROWS_PT, ROWS_PT)])

    @pl.when(c == 1)
    def _():
        pltpu.sync_copy(degsp.at[pl.ds(s * ROWS_PT, ROWS_PT)],
                        out1.at[pl.ds(s * ROWS_PT, ROWS_PT)])


@functools.partial(
    pl.kernel,
    out_type=jax.ShapeDtypeStruct((NP, D), jnp.float32),
    mesh=_mesh2,
    compiler_params=pltpu.CompilerParams(needs_layout_passes=False),
    scratch_types=[
        pltpu.VMEM((BGR, 128), jnp.int32),
        pltpu.VMEM((BGR, 128), jnp.int32),
        pltpu.VMEM((2, 16), jnp.int32),
        pltpu.VMEM((4, 128, D), jnp.float32),
        pltpu.VMEM((ZR_PT, D), jnp.float32),
        pltpu.VMEM_SHARED((NACC, D), jnp.float32),
        pltpu.SemaphoreType.DMA,
        pltpu.SemaphoreType.DMA,
    ],
)
def _agg_kernel(ys_hbm, psrc_hbm, pdst_hbm, pcnt_hbm, out,
                srcr, dstr, crows, rows, zrows, accsp, semg, sems):
    c = lax.axis_index("c")
    s = lax.axis_index("s")
    z16 = jnp.zeros((16,), jnp.float32)
    i16 = lax.iota(jnp.int32, 16)
    pltpu.sync_copy(pcnt_hbm.at[pl.ds(2 * s * 16, 16)], crows.at[0])
    pltpu.sync_copy(pcnt_hbm.at[pl.ds((2 * s + 1) * 16, 16)], crows.at[1])

    def zrow(i, carry):
        for k in range(D // 16):
            zrows[i, pl.ds(k * 16, 16)] = z16
        return carry
    lax.fori_loop(0, ZR_PT, zrow, 0)

    for p in range(P // 2):
        q = 2 * c + p
        pltpu.sync_copy(zrows, accsp.at[pl.ds(s * ZR_PT, ZR_PT)])
        plsc.subcore_barrier()

        for j in range(2):
            gbase = (2 * s + j) * (BPT // 128) + q * BGR
            pltpu.sync_copy(psrc_hbm.at[pl.ds(gbase, BGR)], srcr)
            pltpu.sync_copy(pdst_hbm.at[pl.ds(gbase, BGR)], dstr)
            cnt = jnp.sum(jnp.where(i16 == q, crows[j], 0))
            ngr = (cnt + 127) >> 7

            for t in range(3):
                @pl.when(t < ngr)
                def _():
                    pltpu.async_copy(ys_hbm.at[srcr.at[t]], rows.at[t], semg)

            def grp(g, carry):
                buf = g & 3
                pltpu.make_async_copy(ys_hbm.at[srcr.at[0]],
                                      rows.at[buf], semg).wait()
                pltpu.async_copy(rows.at[buf], accsp.at[dstr.at[g]], sems,
                                 add=True)

                @pl.when(g >= 1)
                def _():
                    pltpu.make_async_copy(rows.at[0], accsp.at[dstr.at[0]],
                                          sems).wait()

                @pl.when(g + 3 < ngr)
                def _():
                    pltpu.async_copy(ys_hbm.at[srcr.at[g + 3]],
                                     rows.at[(g + 3) & 3], semg)
                return carry
            lax.fori_loop(0, ngr, grp, 0)

            @pl.when(ngr >= 1)
            def _():
                pltpu.make_async_copy(rows.at[0], accsp.at[dstr.at[0]],
                                      sems).wait()

        plsc.subcore_barrier()
        pltpu.sync_copy(accsp.at[pl.ds(s * CP_PT, CP_PT)],
                        out.at[pl.ds(q * CH + s * CP_PT, CP_PT)])
        plsc.subcore_barrier()


BR = 1024
NB = NP // BR


def _ka_body(x_ref, w_ref, d0_ref, d1_ref, ys_ref, dis_ref):
    deg = d0_ref[...] + d1_ref[...] + 1.0
    dis = lax.rsqrt(deg)
    y = jnp.dot(x_ref[...], w_ref[...], preferred_element_type=jnp.float32)
    ys_ref[...] = y * dis[:, None]
    dis_ref[...] = dis


def _ka_call(xp, W1, d0, d1):
    return pl.pallas_call(
        _ka_body,
        grid=(NB,),
        in_specs=[
            pl.BlockSpec((BR, D), lambda i: (i, 0)),
            pl.BlockSpec((D, D), lambda i: (0, 0)),
            pl.BlockSpec((BR,), lambda i: (i,)),
            pl.BlockSpec((BR,), lambda i: (i,)),
        ],
        out_specs=[
            pl.BlockSpec((BR, D), lambda i: (i, 0)),
            pl.BlockSpec((BR,), lambda i: (i,)),
        ],
        out_shape=[
            jax.ShapeDtypeStruct((NP, D), jnp.float32),
            jax.ShapeDtypeStruct((NP,), jnp.float32),
        ],
    )(xp, W1, d0, d1)


def _kb_body(a_ref, ys_ref, dis_ref, b_ref, w_ref, ysn_ref):
    dis = dis_ref[...]
    h = (a_ref[...] + ys_ref[...]) * dis[:, None] + b_ref[...][None, :]
    h = jnp.maximum(h, 0.0)
    y2 = jnp.dot(h, w_ref[...], preferred_element_type=jnp.float32)
    ysn_ref[...] = y2 * dis[:, None]


def _kb_call(a, ys, dis, b_t, W_t):
    return pl.pallas_call(
        _kb_body,
        grid=(NB,),
        in_specs=[
            pl.BlockSpec((BR, D), lambda i: (i, 0)),
            pl.BlockSpec((BR, D), lambda i: (i, 0)),
            pl.BlockSpec((BR,), lambda i: (i,)),
            pl.BlockSpec((D,), lambda i: (0,)),
            pl.BlockSpec((D, D), lambda i: (0, 0)),
        ],
        out_specs=pl.BlockSpec((BR, D), lambda i: (i, 0)),
        out_shape=jax.ShapeDtypeStruct((NP, D), jnp.float32),
    )(a, ys, dis, b_t, W_t)


def _kc_body(a_ref, ys2_ref, dis_ref, b_ref, bat_ref, wfc_ref,
             bfc_ref, out_ref, sums, cnt):
    i = pl.program_id(0)

    @pl.when(i == 0)
    def _():
        sums[...] = jnp.zeros_like(sums)
        cnt[...] = jnp.zeros_like(cnt)

    dis = dis_ref[...]
    h = (a_ref[...] + ys2_ref[...]) * dis[:, None] + b_ref[...][None, :]
    h = jnp.maximum(h, 0.0)
    bb = bat_ref[...]
    gid = lax.broadcasted_iota(jnp.int32, (BR, G), 1).astype(jnp.float32)
    oh = (bb[:, None] == gid).astype(jnp.float32)
    dn = (((0,), (0,)), ((), ()))
    sums[...] += lax.dot_general(oh, h, dn,
                                 preferred_element_type=jnp.float32)
    cnt[...] += lax.dot_general(oh, jnp.ones((BR, D), jnp.float32), dn,
                                preferred_element_type=jnp.float32)

    @pl.when(i == pl.num_programs(0) - 1)
    def _():
        pooled = sums[...] / jnp.maximum(cnt[...], 1.0)
        o = jnp.dot(pooled, wfc_ref[...], preferred_element_type=jnp.float32)
        out_ref[...] = jnp.maximum(o + bfc_ref[...][None, :], 0.0)


def _kc_call(a2, ys2, dis, b2, batf, Wfc, bfc):
    return pl.pallas_call(
        _kc_body,
        grid=(NB,),
        in_specs=[
            pl.BlockSpec((BR, D), lambda i: (i, 0)),
            pl.BlockSpec((BR, D), lambda i: (i, 0)),
            pl.BlockSpec((BR,), lambda i: (i,)),
            pl.BlockSpec((D,), lambda i: (0,)),
            pl.BlockSpec((BR,), lambda i: (i,)),
            pl.BlockSpec((D, D), lambda i: (0, 0)),
            pl.BlockSpec((D,), lambda i: (0,)),
        ],
        out_specs=pl.BlockSpec((G, D), lambda i: (0, 0)),
        out_shape=jax.ShapeDtypeStruct((G, D), jnp.float32),
        scratch_shapes=[
            pltpu.VMEM((G, D), jnp.float32),
            pltpu.VMEM((G, D), jnp.float32),
        ],
    )(a2, ys2, dis, b2, batf, Wfc, bfc)


def kernel(x, edge_index, batch, W1, b1, W2, b2, Wfc, bfc):
    src = edge_index[0].astype(jnp.int32)
    dst = edge_index[1].astype(jnp.int32)
    npad = EP - E
    pad_idx = N + (jnp.arange(npad, dtype=jnp.int32) % (NP - N))
    src2 = jnp.concatenate([src, pad_idx]).reshape(NGR, 128)
    dst2 = jnp.concatenate([dst, pad_idx]).reshape(NGR, 128)
    xp = jnp.pad(x, ((0, NP - N), (0, 0)))
    batf = jnp.pad(batch.astype(jnp.float32), (0, NP - N),
                   constant_values=float(G))

    d0, d1, psrc, pdst, pcnt = _deg_part_kernel(src2, dst2)
    ys1, dis = _ka_call(xp, W1, d0, d1)
    a1 = _agg_kernel(ys1, psrc, pdst, pcnt)
    ys2 = _kb_call(a1, ys1, dis, b1, W2)
    a2 = _agg_kernel(ys2, psrc, pdst, pcnt)
    return _kc_call(a2, ys2, dis, b2, batf, Wfc, bfc)


# speedup vs baseline: 34.2870x; 1.0221x over previous
import functools

import jax
import jax.numpy as jnp
from jax import lax
from jax.experimental import pallas as pl
from jax.experimental.pallas import tpu as pltpu
from jax.experimental.pallas import tpu_sc as plsc

N = 10000
NP = 10240
P = 4
CH = 2560
TRASH = 128
NACC = CH + TRASH
E = 320000
D = 128
G = 64
NS = 16
NT = 2 * NS
EP = 327680
NGR = EP // 128
GPT = NGR // NT
BCAP = 4096
BGR = BCAP // 128
BPT = P * BCAP
ROWS_PT = NP // NS
ZR_PT = NACC // NS
CP_PT = CH // NS

_mesh2 = plsc.VectorSubcoreMesh(core_axis_name="c", subcore_axis_name="s")
_mesh1 = plsc.VectorSubcoreMesh(core_axis_name="c", subcore_axis_name="s",
                                num_cores=1)


@functools.partial(
    pl.kernel,
    out_type=(jax.ShapeDtypeStruct((NP,), jnp.float32),
              jax.ShapeDtypeStruct((NP,), jnp.float32),
              jax.ShapeDtypeStruct((NT * BPT // 128, 128), jnp.int32),
              jax.ShapeDtypeStruct((NT * BPT // 128, 128), jnp.int32),
              jax.ShapeDtypeStruct((NT * 16,), jnp.int32)),
    mesh=_mesh2,
    compiler_params=pltpu.CompilerParams(needs_layout_passes=False),
    scratch_types=[
        pltpu.VMEM((GPT, 128), jnp.int32),
        pltpu.VMEM((GPT, 128), jnp.int32),
        pltpu.VMEM((BPT // 128, 128), jnp.int32),
        pltpu.VMEM((BPT // 128, 128), jnp.int32),
        pltpu.VMEM((16,), jnp.int32),
        pltpu.VMEM((128,), jnp.float32),
        pltpu.VMEM((ROWS_PT,), jnp.float32),
        pltpu.VMEM_SHARED((NP,), jnp.float32),
        pltpu.SemaphoreType.DMA,
    ],
)
def _deg_part_kernel(src2_hbm, dst2_hbm, out0, out1, psrc, pdst, pcnt,
                     srcb, dstb, bsrc, bdst, cntb, onesb, zbuf, degsp, semh):
    c = lax.axis_index("c")
    s = lax.axis_index("s")
    wid = c * NS + s
    z16 = jnp.zeros((16,), jnp.float32)
    o16 = jnp.ones((16,), jnp.float32)
    i16 = lax.iota(jnp.int32, 16)
    for k in range(128 // 16):
        onesb[pl.ds(k * 16, 16)] = o16
    for k in range(ROWS_PT // 16):
        zbuf[pl.ds(k * 16, 16)] = z16
    pltpu.sync_copy(zbuf, degsp.at[pl.ds(s * ROWS_PT, ROWS_PT)])

    pltpu.sync_copy(src2_hbm.at[pl.ds(wid * GPT, GPT)], srcb)
    pltpu.sync_copy(dst2_hbm.at[pl.ds(wid * GPT, GPT)], dstb)

    def fill(i, carry):
        r = i >> 3
        k = i & 7
        pat = (i16 + i * 16) & (TRASH - 1)
        bsrc[r, pl.ds(k * 16, 16)] = N + pat
        bdst[r, pl.ds(k * 16, 16)] = CH + pat
        return carry
    lax.fori_loop(0, BPT // 16, fill, 0)

    plsc.subcore_barrier()
    for g in range(GPT):
        pltpu.async_copy(onesb, degsp.at[dstb.at[g]], semh, add=True)

    def part(i, offs):
        g = i >> 3
        k = i & 7
        dv = dstb[g, pl.ds(k * 16, 16)]
        sv = srcb[g, pl.ds(k * 16, 16)]
        new_offs = []
        for q in range(P):
            inq = (dv >= q * CH) & (dv < (q + 1) * CH)
            cs = plsc.cumsum(inq.astype(jnp.int32))
            pos = q * BCAP + offs[q] + cs - 1
            ok = inq & (pos < (q + 1) * BCAP)
            plsc.store_scatter(bsrc, [pos >> 7, pos & 127], sv, mask=ok)
            plsc.store_scatter(bdst, [pos >> 7, pos & 127], dv - q * CH,
                               mask=ok)
            new_offs.append(offs[q] + cs[15])
        return tuple(new_offs)
    zero = jnp.zeros((), jnp.int32)
    offs = lax.fori_loop(0, GPT * 8, part, (zero, zero, zero, zero))
    cvec = jnp.zeros((16,), jnp.int32)
    for q in range(P):
        cvec = jnp.where(i16 == q, jnp.minimum(offs[q], BCAP), cvec)
    cntb[...] = cvec
    pltpu.sync_copy(cntb, pcnt.at[pl.ds(wid * 16, 16)])
    for g in range(GPT):
        pltpu.make_async_copy(onesb, degsp.at[dstb.at[g]], semh).wait()

    gpr = BPT // 128
    pltpu.sync_copy(bsrc, psrc.at[pl.ds(wid * gpr, gpr)])
    pltpu.sync_copy(bdst, pdst.at[pl.ds(wid * gpr, gpr)])

    plsc.subcore_barrier()

    @pl.when(c == 0)
    def _():
        pltpu.sync_copy(degsp.at[pl.ds(s * ROWS_PT, ROWS_PT)],
                        out0.at[pl.ds(s * ROWS_PT, ROWS_PT)])

    @pl.when(c == 1)
    def _():
        pltpu.sync_copy(degsp.at[pl.ds(s * ROWS_PT, ROWS_PT)],
                        out1.at[pl.ds(s * ROWS_PT, ROWS_PT)])


@functools.partial(
    pl.kernel,
    out_type=jax.ShapeDtypeStruct((NP, D), jnp.float32),
    mesh=_mesh2,
    compiler_params=pltpu.CompilerParams(needs_layout_passes=False),
    scratch_types=[
        pltpu.VMEM((BGR, 128), jnp.int32),
        pltpu.VMEM((BGR, 128), jnp.int32),
        pltpu.VMEM((2, 16), jnp.int32),
        pltpu.VMEM((4, 128, D), jnp.float32),
        pltpu.VMEM((ZR_PT, D), jnp.float32),
        pltpu.VMEM_SHARED((NACC, D), jnp.float32),
        pltpu.SemaphoreType.DMA,
        pltpu.SemaphoreType.DMA,
    ],
)
def _agg_kernel(ys_hbm, psrc_hbm, pdst_hbm, pcnt_hbm, out,
                srcr, dstr, crows, rows, zrows, accsp, semg, sems):
    c = lax.axis_index("c")
    s = lax.axis_index("s")
    z16 = jnp.zeros((16,), jnp.float32)
    i16 = lax.iota(jnp.int32, 16)
    pltpu.sync_copy(pcnt_hbm.at[pl.ds(2 * s * 16, 16)], crows.at[0])
    pltpu.sync_copy(pcnt_hbm.at[pl.ds((2 * s + 1) * 16, 16)], crows.at[1])

    def zrow(i, carry):
        for k in range(D // 16):
            zrows[i, pl.ds(k * 16, 16)] = z16
        return carry
    lax.fori_loop(0, ZR_PT, zrow, 0)

    for p in range(P // 2):
        q = 2 * c + p
        pltpu.sync_copy(zrows, accsp.at[pl.ds(s * ZR_PT, ZR_PT)])
        plsc.subcore_barrier()

        for j in range(2):
            gbase = (2 * s + j) * (BPT // 128) + q * BGR
            pltpu.sync_copy(psrc_hbm.at[pl.ds(gbase, BGR)], srcr)
            pltpu.sync_copy(pdst_hbm.at[pl.ds(gbase, BGR)], dstr)
            cnt = jnp.sum(jnp.where(i16 == q, crows[j], 0))
            ngr = (cnt + 127) >> 7

            for t in range(3):
                @pl.when(t < ngr)
                def _():
                    pltpu.async_copy(ys_hbm.at[srcr.at[t]], rows.at[t], semg)

            def grp(g, carry):
                buf = g & 3
                pltpu.make_async_copy(ys_hbm.at[srcr.at[0]],
                                      rows.at[buf], semg).wait()
                pltpu.async_copy(rows.at[buf], accsp.at[dstr.at[g]], sems,
                                 add=True)

                @pl.when(g >= 1)
                def _():
                    pltpu.make_async_copy(rows.at[0], accsp.at[dstr.at[0]],
                                          sems).wait()

                @pl.when(g + 3 < ngr)
                def _():
                    pltpu.async_copy(ys_hbm.at[srcr.at[g + 3]],
                                     rows.at[(g + 3) & 3], semg)
                return carry
            lax.fori_loop(0, ngr, grp, 0)

            @pl.when(ngr >= 1)
            def _():
                pltpu.make_async_copy(rows.at[0], accsp.at[dstr.at[0]],
                                      sems).wait()

        plsc.subcore_barrier()
        pltpu.sync_copy(accsp.at[pl.ds(s * CP_PT, CP_PT)],
                        out.at[pl.ds(q * CH + s * CP_PT, CP_PT)])
        plsc.subcore_barrier()


BR = 1024
NB = NP // BR


def _ka1_body(x_ref, w_ref, y_ref):
    y_ref[...] = jnp.dot(x_ref[...], w_ref[...],
                         preferred_element_type=jnp.float32)


def _ka1_call(xp, W1):
    return pl.pallas_call(
        _ka1_body,
        grid=(NB,),
        in_specs=[
            pl.BlockSpec((BR, D), lambda i: (i, 0)),
            pl.BlockSpec((D, D), lambda i: (0, 0)),
        ],
        out_specs=pl.BlockSpec((BR, D), lambda i: (i, 0)),
        out_shape=jax.ShapeDtypeStruct((NP, D), jnp.float32),
    )(xp, W1)


def _ka2_body(y_ref, d0_ref, d1_ref, ys_ref, dis_ref):
    deg = d0_ref[...] + d1_ref[...] + 1.0
    dis = lax.rsqrt(deg)
    ys_ref[...] = y_ref[...] * dis[:, None]
    dis_ref[...] = dis


def _ka2_call(y1, d0, d1):
    return pl.pallas_call(
        _ka2_body,
        grid=(NB,),
        in_specs=[
            pl.BlockSpec((BR, D), lambda i: (i, 0)),
            pl.BlockSpec((BR,), lambda i: (i,)),
            pl.BlockSpec((BR,), lambda i: (i,)),
        ],
        out_specs=[
            pl.BlockSpec((BR, D), lambda i: (i, 0)),
            pl.BlockSpec((BR,), lambda i: (i,)),
        ],
        out_shape=[
            jax.ShapeDtypeStruct((NP, D), jnp.float32),
            jax.ShapeDtypeStruct((NP,), jnp.float32),
        ],
    )(y1, d0, d1)


def _kb_body(a_ref, ys_ref, dis_ref, b_ref, w_ref, ysn_ref):
    dis = dis_ref[...]
    h = (a_ref[...] + ys_ref[...]) * dis[:, None] + b_ref[...][None, :]
    h = jnp.maximum(h, 0.0)
    y2 = jnp.dot(h, w_ref[...], preferred_element_type=jnp.float32)
    ysn_ref[...] = y2 * dis[:, None]


def _kb_call(a, ys, dis, b_t, W_t):
    return pl.pallas_call(
        _kb_body,
        grid=(NB,),
        in_specs=[
            pl.BlockSpec((BR, D), lambda i: (i, 0)),
            pl.BlockSpec((BR, D), lambda i: (i, 0)),
            pl.BlockSpec((BR,), lambda i: (i,)),
            pl.BlockSpec((D,), lambda i: (0,)),
            pl.BlockSpec((D, D), lambda i: (0, 0)),
        ],
        out_specs=pl.BlockSpec((BR, D), lambda i: (i, 0)),
        out_shape=jax.ShapeDtypeStruct((NP, D), jnp.float32),
    )(a, ys, dis, b_t, W_t)


def _kc_body(a_ref, ys2_ref, dis_ref, b_ref, bat_ref, wfc_ref,
             bfc_ref, out_ref, sums, cnt):
    i = pl.program_id(0)

    @pl.when(i == 0)
    def _():
        sums[...] = jnp.zeros_like(sums)
        cnt[...] = jnp.zeros_like(cnt)

    dis = dis_ref[...]
    h = (a_ref[...] + ys2_ref[...]) * dis[:, None] + b_ref[...][None, :]
    h = jnp.maximum(h, 0.0)
    bb = bat_ref[...]
    gid = lax.broadcasted_iota(jnp.int32, (BR, G), 1).astype(jnp.float32)
    oh = (bb[:, None] == gid).astype(jnp.float32)
    dn = (((0,), (0,)), ((), ()))
    sums[...] += lax.dot_general(oh, h, dn,
                                 preferred_element_type=jnp.float32)
    cnt[...] += lax.dot_general(oh, jnp.ones((BR, D), jnp.float32), dn,
                                preferred_element_type=jnp.float32)

    @pl.when(i == pl.num_programs(0) - 1)
    def _():
        pooled = sums[...] / jnp.maximum(cnt[...], 1.0)
        o = jnp.dot(pooled, wfc_ref[...], preferred_element_type=jnp.float32)
        out_ref[...] = jnp.maximum(o + bfc_ref[...][None, :], 0.0)


def _kc_call(a2, ys2, dis, b2, batf, Wfc, bfc):
    return pl.pallas_call(
        _kc_body,
        grid=(NB,),
        in_specs=[
            pl.BlockSpec((BR, D), lambda i: (i, 0)),
            pl.BlockSpec((BR, D), lambda i: (i, 0)),
            pl.BlockSpec((BR,), lambda i: (i,)),
            pl.BlockSpec((D,), lambda i: (0,)),
            pl.BlockSpec((BR,), lambda i: (i,)),
            pl.BlockSpec((D, D), lambda i: (0, 0)),
            pl.BlockSpec((D,), lambda i: (0,)),
        ],
        out_specs=pl.BlockSpec((G, D), lambda i: (0, 0)),
        out_shape=jax.ShapeDtypeStruct((G, D), jnp.float32),
        scratch_shapes=[
            pltpu.VMEM((G, D), jnp.float32),
            pltpu.VMEM((G, D), jnp.float32),
        ],
    )(a2, ys2, dis, b2, batf, Wfc, bfc)


def kernel(x, edge_index, batch, W1, b1, W2, b2, Wfc, bfc):
    src = edge_index[0].astype(jnp.int32)
    dst = edge_index[1].astype(jnp.int32)
    npad = EP - E
    pad_idx = N + (jnp.arange(npad, dtype=jnp.int32) % (NP - N))
    src2 = jnp.concatenate([src, pad_idx]).reshape(NGR, 128)
    dst2 = jnp.concatenate([dst, pad_idx]).reshape(NGR, 128)
    xp = jnp.pad(x, ((0, NP - N), (0, 0)))
    batf = jnp.pad(batch.astype(jnp.float32), (0, NP - N),
                   constant_values=float(G))

    d0, d1, psrc, pdst, pcnt = _deg_part_kernel(src2, dst2)
    y1 = _ka1_call(xp, W1)
    ys1, dis = _ka2_call(y1, d0, d1)
    a1 = _agg_kernel(ys1, psrc, pdst, pcnt)
    ys2 = _kb_call(a1, ys1, dis, b1, W2)
    a2 = _agg_kernel(ys2, psrc, pdst, pcnt)
    return _kc_call(a2, ys2, dis, b2, batf, Wfc, bfc)
